# Initial kernel scaffold; baseline (speedup 1.0000x reference)
import jax
import jax.numpy as jnp
from jax.experimental import pallas as pl


def kernel(h, edge_index, edge_attr, W1, b1, W2, b2, gamma, beta):
    raise NotImplementedError("write your pallas kernel here")


# speedup vs baseline: 3.2125x; 3.2125x over previous
import functools

import jax
import jax.numpy as jnp
from jax import lax
from jax.experimental import pallas as pl
from jax.experimental.pallas import tpu as pltpu
from jax.experimental.pallas import tpu_sc as plsc

NC = 2
NS = 16
NW = NC * NS
CH = 80


def _tc_premul(h, W1a):
    N, D = h.shape
    BN = 2000

    def body(h_ref, w_ref, o_ref):
        o_ref[...] = jnp.dot(h_ref[...], w_ref[...],
                             preferred_element_type=jnp.float32)

    return pl.pallas_call(
        body,
        grid=(N // BN,),
        in_specs=[
            pl.BlockSpec((BN, D), lambda i: (i, 0)),
            pl.BlockSpec((D, D), lambda i: (0, 0)),
        ],
        out_specs=pl.BlockSpec((BN, D), lambda i: (i, 0)),
        out_shape=jax.ShapeDtypeStruct((N, D), jnp.float32),
    )(h, W1a)


def _sc_gather(hW, src):
    N, D = hW.shape
    E = src.shape[0]
    EPW = E // NW
    steps = EPW // CH
    mesh = plsc.VectorSubcoreMesh(core_axis_name="c", subcore_axis_name="s")

    @functools.partial(
        pl.kernel,
        out_type=jax.ShapeDtypeStruct((E, D), jnp.float32),
        mesh=mesh,
        scratch_types=[
            pltpu.VMEM((CH,), jnp.int32),
            pltpu.VMEM((CH, D), jnp.float32),
            pltpu.SemaphoreType.DMA,
        ],
    )
    def k(hW_hbm, src_hbm, out_hbm, idx_v, rows_v, sem):
        cid = lax.axis_index("c")
        sid = lax.axis_index("s")
        wid = sid * NC + cid

        @pl.loop(0, steps)
        def _(i):
            base = wid * EPW + i * CH
            pltpu.sync_copy(src_hbm.at[pl.ds(base, CH)], idx_v)
            pltpu.async_copy(hW_hbm.at[idx_v], rows_v, sem).wait()
            pltpu.sync_copy(rows_v, out_hbm.at[pl.ds(base, CH)])

    return k(hW, src)


def _tc_mlp(gathered, eaT, W1b, b1, W2, b2):
    E, D = gathered.shape
    F = eaT.shape[0]
    BE = 2560

    def body(g_ref, ea_ref, w1b_ref, b1_ref, w2_ref, b2_ref, o_ref):
        pre = lax.dot_general(ea_ref[...], w1b_ref[...],
                              (((0,), (0,)), ((), ())),
                              preferred_element_type=jnp.float32)
        x = jnp.maximum(g_ref[...] + pre + b1_ref[...], 0.0)
        o_ref[...] = jnp.dot(x, w2_ref[...],
                             preferred_element_type=jnp.float32) + b2_ref[...]

    return pl.pallas_call(
        body,
        grid=(E // BE,),
        in_specs=[
            pl.BlockSpec((BE, D), lambda i: (i, 0)),
            pl.BlockSpec((F, BE), lambda i: (0, i)),
            pl.BlockSpec((F, D), lambda i: (0, 0)),
            pl.BlockSpec((1, D), lambda i: (0, 0)),
            pl.BlockSpec((D, D), lambda i: (0, 0)),
            pl.BlockSpec((1, D), lambda i: (0, 0)),
        ],
        out_specs=pl.BlockSpec((BE, D), lambda i: (i, 0)),
        out_shape=jax.ShapeDtypeStruct((E, D), jnp.float32),
    )(gathered, eaT, W1b, b1, W2, b2)


def _sc_scatter(msg, dst, zidx, N):
    E, D = msg.shape
    EPW = E // NW
    steps = EPW // CH
    NPAD = 10240
    CR = NPAD // 128
    ZR = NPAD // NS
    ZB = 80
    mesh = plsc.VectorSubcoreMesh(core_axis_name="c", subcore_axis_name="s")

    @functools.partial(
        pl.kernel,
        out_type=(
            jax.ShapeDtypeStruct((NC * NPAD, D), jnp.float32),
            jax.ShapeDtypeStruct((NC * CR, 128), jnp.float32),
        ),
        mesh=mesh,
        compiler_params=pltpu.CompilerParams(needs_layout_passes=False),
        scratch_types=[
            pltpu.VMEM_SHARED((NPAD, D), jnp.float32),
            pltpu.VMEM_SHARED((CR, 128), jnp.float32),
            pltpu.VMEM((CH,), jnp.int32),
            pltpu.VMEM((CH, D), jnp.float32),
            pltpu.VMEM((ZB,), jnp.int32),
            pltpu.VMEM((ZB, D), jnp.float32),
            pltpu.VMEM((NPAD,), jnp.float32),
            pltpu.VMEM((CR, 128), jnp.float32),
            pltpu.SemaphoreType.DMA,
        ],
    )
    def k(msg_hbm, dst_hbm, zidx_hbm, aggp_hbm, cntp_hbm,
          agg_sh, cnt_sh, idx_v, rows_v, idx_w, zbuf, cnt_f, cnt_l, sem):
        cid = lax.axis_index("c")
        sid = lax.axis_index("s")
        wid = sid * NC + cid
        zv = jnp.zeros((16,), jnp.float32)
        ov = jnp.full((16,), 1.0, jnp.float32)

        @pl.loop(0, ZB)
        def _(r):
            for j in range(D // 16):
                zbuf[r, pl.ds(j * 16, 16)] = zv

        @pl.loop(0, CR)
        def _(r):
            for j in range(128 // 16):
                cnt_f[pl.ds(r * 128 + j * 16, 16)] = zv

        @pl.loop(0, ZR // ZB)
        def _(t):
            base = sid * ZR + t * ZB
            pltpu.sync_copy(zidx_hbm.at[pl.ds(base, ZB)], idx_w)
            pltpu.sync_copy(zbuf, agg_sh.at[idx_w])

        @pl.when(sid == 0)
        def _():
            pltpu.sync_copy(zidx_hbm.at[pl.ds(0, ZB)], idx_w)
            pltpu.sync_copy(zbuf, cnt_sh.at[idx_w])

        plsc.subcore_barrier()

        @pl.loop(0, steps)
        def _(i):
            base = wid * EPW + i * CH
            pltpu.sync_copy(dst_hbm.at[pl.ds(base, CH)], idx_v)
            pltpu.async_copy(msg_hbm.at[pl.ds(base, CH)], rows_v, sem).wait()
            pltpu.sync_copy(rows_v, agg_sh.at[idx_v], add=True)
            for kk in range(CH // 16):
                d16 = idx_v[pl.ds(kk * 16, 16)]
                plsc.addupdate_scatter(cnt_f, [d16], ov)

        @pl.loop(0, CR)
        def _(r):
            for j in range(128 // 16):
                cnt_l[r, pl.ds(j * 16, 16)] = cnt_f[pl.ds(r * 128 + j * 16, 16)]

        pltpu.sync_copy(zidx_hbm.at[pl.ds(0, CR)], idx_w)
        pltpu.sync_copy(cnt_l, cnt_sh.at[idx_w], add=True)

        plsc.subcore_barrier()

        @pl.loop(0, ZR // ZB)
        def _(t):
            base = sid * ZR + t * ZB
            pltpu.sync_copy(zidx_hbm.at[pl.ds(base, ZB)], idx_w)
            pltpu.sync_copy(agg_sh.at[idx_w], zbuf)
            pltpu.sync_copy(zbuf, aggp_hbm.at[pl.ds(cid * NPAD + base, ZB)])

        @pl.when(sid == 0)
        def _():
            pltpu.sync_copy(zidx_hbm.at[pl.ds(0, CR)], idx_w)
            pltpu.sync_copy(cnt_sh.at[idx_w], cnt_l)
            pltpu.sync_copy(cnt_l, cntp_hbm.at[pl.ds(cid * CR, CR)])

    return k(msg, dst, zidx)


def _tc_norm(h, aggp, cntp, gamma, beta):
    N, D = h.shape
    BN = 2000
    CW = cntp.shape[-1]

    def body(h_ref, a_ref, c_ref, g_ref, b_ref, o_ref):
        agg = a_ref[0] + a_ref[1]
        cnt = c_ref[0] + c_ref[1] + 1.0
        y = h_ref[...] + agg / cnt
        mean = jnp.mean(y, axis=-1, keepdims=True)
        var = jnp.mean((y - mean) ** 2, axis=-1, keepdims=True)
        o_ref[...] = (y - mean) * lax.rsqrt(var + 1e-5) * g_ref[...] + b_ref[...]

    return pl.pallas_call(
        body,
        grid=(N // BN,),
        in_specs=[
            pl.BlockSpec((BN, D), lambda i: (i, 0)),
            pl.BlockSpec((NC, BN, D), lambda i: (0, i, 0)),
            pl.BlockSpec((NC, BN, CW), lambda i: (0, i, 0)),
            pl.BlockSpec((1, D), lambda i: (0, 0)),
            pl.BlockSpec((1, D), lambda i: (0, 0)),
        ],
        out_specs=pl.BlockSpec((BN, D), lambda i: (i, 0)),
        out_shape=jax.ShapeDtypeStruct((N, D), jnp.float32),
    )(h, aggp, cntp, gamma, beta)


def kernel(h, edge_index, edge_attr, W1, b1, W2, b2, gamma, beta):
    N, D = h.shape
    src = edge_index[0]
    dst = edge_index[1]
    W1a = W1[:D]
    W1b = W1[D:]
    eaT = edge_attr.T
    b1r = b1.reshape(1, D)
    b2r = b2.reshape(1, D)

    hW = _tc_premul(h, W1a)
    gathered = _sc_gather(hW, src)
    msg = _tc_mlp(gathered, eaT, W1b, b1r, W2, b2r)
    zidx = jnp.arange(10240, dtype=jnp.int32)
    aggp, cntp = _sc_scatter(msg, dst, zidx, N)
    aggp = aggp.reshape(NC, -1, D)[:, :N]
    cntp = cntp.reshape(NC, -1)[:, :N].reshape(NC, N, 1)
    return _tc_norm(h, aggp, cntp, gamma.reshape(1, D), beta.reshape(1, D))


# speedup vs baseline: 4.6182x; 1.4376x over previous
import functools

import jax
import jax.numpy as jnp
from jax import lax
from jax.experimental import pallas as pl
from jax.experimental.pallas import tpu as pltpu
from jax.experimental.pallas import tpu_sc as plsc

NC = 2
NS = 16
NW = NC * NS
CH = 80


def _tc_premul(h, W1a):
    N, D = h.shape
    BN = 2000

    def body(h_ref, w_ref, o_ref):
        o_ref[...] = jnp.dot(h_ref[...], w_ref[...],
                             preferred_element_type=jnp.float32)

    return pl.pallas_call(
        body,
        grid=(N // BN,),
        in_specs=[
            pl.BlockSpec((BN, D), lambda i: (i, 0)),
            pl.BlockSpec((D, D), lambda i: (0, 0)),
        ],
        out_specs=pl.BlockSpec((BN, D), lambda i: (i, 0)),
        out_shape=jax.ShapeDtypeStruct((N, D), jnp.float32),
    )(h, W1a)


def _sc_gather(hW, src):
    N, D = hW.shape
    E = src.shape[0]
    EPW = E // NW
    steps = EPW // CH
    mesh = plsc.VectorSubcoreMesh(core_axis_name="c", subcore_axis_name="s")

    @functools.partial(
        pl.kernel,
        out_type=jax.ShapeDtypeStruct((E, D), jnp.float32),
        mesh=mesh,
        scratch_types=[
            pltpu.VMEM((CH,), jnp.int32),
            pltpu.VMEM((CH,), jnp.int32),
            pltpu.VMEM((CH, D), jnp.float32),
            pltpu.VMEM((CH, D), jnp.float32),
            pltpu.SemaphoreType.DMA,
            pltpu.SemaphoreType.DMA,
            pltpu.SemaphoreType.DMA,
            pltpu.SemaphoreType.DMA,
            pltpu.SemaphoreType.DMA,
            pltpu.SemaphoreType.DMA,
        ],
    )
    def k(hW_hbm, src_hbm, out_hbm, idx0, idx1, rows0, rows1,
          si0, si1, sg0, sg1, so0, so1):
        cid = lax.axis_index("c")
        sid = lax.axis_index("s")
        wid = sid * NC + cid
        idxs = (idx0, idx1)
        rows = (rows0, rows1)
        isem = (si0, si1)
        gsem = (sg0, sg1)
        osem = (so0, so1)

        def istart(i, b):
            base = wid * EPW + i * CH
            pltpu.async_copy(src_hbm.at[pl.ds(base, CH)], idxs[b], isem[b])

        def iwait(b):
            pltpu.make_async_copy(src_hbm.at[pl.ds(0, CH)], idxs[b],
                                  isem[b]).wait()

        def owait(b):
            pltpu.make_async_copy(rows[b], out_hbm.at[pl.ds(0, CH)],
                                  osem[b]).wait()

        def run(i, b, first, last):
            iwait(b)
            if not first:
                owait(b)
            pltpu.async_copy(hW_hbm.at[idxs[b]], rows[b], gsem[b]).wait()
            base = wid * EPW + i * CH
            pltpu.async_copy(rows[b], out_hbm.at[pl.ds(base, CH)], osem[b])
            if not last:
                @pl.when(i + 2 < steps)
                def _():
                    istart(i + 2, b)

        istart(0, 0)
        istart(1, 1)
        run(0, 0, first=True, last=False)
        run(1, 1, first=True, last=False)

        @pl.loop(1, steps // 2)
        def _(p):
            for b in (0, 1):
                run(2 * p + b, b, first=False, last=False)

        if steps % 2:
            run(steps - 1, 0, first=False, last=True)
        owait(steps % 2)
        owait(1 - steps % 2)

    return k(hW, src)


def _tc_mlp(gathered, eaT, W1b, b1, W2, b2):
    E, D = gathered.shape
    F = eaT.shape[0]
    BE = 2560

    def body(g_ref, ea_ref, w1b_ref, b1_ref, w2_ref, b2_ref, o_ref):
        pre = lax.dot_general(ea_ref[...], w1b_ref[...],
                              (((0,), (0,)), ((), ())),
                              preferred_element_type=jnp.float32)
        x = jnp.maximum(g_ref[...] + pre + b1_ref[...], 0.0)
        o_ref[...] = jnp.dot(x, w2_ref[...],
                             preferred_element_type=jnp.float32) + b2_ref[...]

    return pl.pallas_call(
        body,
        grid=(E // BE,),
        in_specs=[
            pl.BlockSpec((BE, D), lambda i: (i, 0)),
            pl.BlockSpec((F, BE), lambda i: (0, i)),
            pl.BlockSpec((F, D), lambda i: (0, 0)),
            pl.BlockSpec((1, D), lambda i: (0, 0)),
            pl.BlockSpec((D, D), lambda i: (0, 0)),
            pl.BlockSpec((1, D), lambda i: (0, 0)),
        ],
        out_specs=pl.BlockSpec((BE, D), lambda i: (i, 0)),
        out_shape=jax.ShapeDtypeStruct((E, D), jnp.float32),
    )(gathered, eaT, W1b, b1, W2, b2)


def _sc_scatter(msg, dst, zidx, N):
    E, D = msg.shape
    EPW = E // NW
    steps = EPW // CH
    NPAD = 10240
    CR = NPAD // 128
    ZR = NPAD // NS
    ZB = 80
    mesh = plsc.VectorSubcoreMesh(core_axis_name="c", subcore_axis_name="s")

    @functools.partial(
        pl.kernel,
        out_type=(
            jax.ShapeDtypeStruct((NC * NPAD, D), jnp.float32),
            jax.ShapeDtypeStruct((NC * CR, 128), jnp.float32),
        ),
        mesh=mesh,
        compiler_params=pltpu.CompilerParams(needs_layout_passes=False),
        scratch_types=[
            pltpu.VMEM_SHARED((NPAD, D), jnp.float32),
            pltpu.VMEM_SHARED((CR, 128), jnp.float32),
            pltpu.VMEM((CH,), jnp.int32),
            pltpu.VMEM((CH,), jnp.int32),
            pltpu.VMEM((CH, D), jnp.float32),
            pltpu.VMEM((CH, D), jnp.float32),
            pltpu.VMEM((ZB,), jnp.int32),
            pltpu.VMEM((NPAD,), jnp.float32),
            pltpu.VMEM((CR, 128), jnp.float32),
            pltpu.SemaphoreType.DMA,
            pltpu.SemaphoreType.DMA,
            pltpu.SemaphoreType.DMA,
            pltpu.SemaphoreType.DMA,
            pltpu.SemaphoreType.DMA,
            pltpu.SemaphoreType.DMA,
            pltpu.SemaphoreType.DMA,
        ],
    )
    def k(msg_hbm, dst_hbm, zidx_hbm, aggp_hbm, cntp_hbm,
          agg_sh, cnt_sh, idx0, idx1, rows0, rows1, idx_w, cnt_f,
          cnt_l, sem, ii0, ii1, mm0, mm1, aa0, aa1):
        cid = lax.axis_index("c")
        sid = lax.axis_index("s")
        wid = sid * NC + cid
        zv = jnp.zeros((16,), jnp.float32)
        ov = jnp.full((16,), 1.0, jnp.float32)

        @pl.loop(0, ZB)
        def _(r):
            for j in range(D // 16):
                rows0[r, pl.ds(j * 16, 16)] = zv

        @pl.loop(0, CR)
        def _(r):
            for j in range(128 // 16):
                cnt_f[pl.ds(r * 128 + j * 16, 16)] = zv

        @pl.loop(0, ZR // ZB)
        def _(t):
            base = sid * ZR + t * ZB
            pltpu.sync_copy(zidx_hbm.at[pl.ds(base, ZB)], idx_w)
            pltpu.sync_copy(rows0, agg_sh.at[idx_w])

        @pl.when(sid == 0)
        def _():
            pltpu.sync_copy(zidx_hbm.at[pl.ds(0, ZB)], idx_w)
            pltpu.sync_copy(rows0, cnt_sh.at[idx_w])

        plsc.subcore_barrier()

        idxs = (idx0, idx1)
        rows = (rows0, rows1)
        iis = (ii0, ii1)
        mms = (mm0, mm1)
        aas = (aa0, aa1)

        def fstart(i, b):
            base = wid * EPW + i * CH
            pltpu.async_copy(dst_hbm.at[pl.ds(base, CH)], idxs[b], iis[b])
            pltpu.async_copy(msg_hbm.at[pl.ds(base, CH)], rows[b], mms[b])

        def runs(i, b, last):
            pltpu.make_async_copy(dst_hbm.at[pl.ds(0, CH)], idxs[b],
                                  iis[b]).wait()
            pltpu.make_async_copy(msg_hbm.at[pl.ds(0, CH)], rows[b],
                                  mms[b]).wait()
            pltpu.async_copy(rows[b], agg_sh.at[idxs[b]], aas[b], add=True)
            for kk in range(CH // 16):
                d16 = idxs[b][pl.ds(kk * 16, 16)]
                plsc.addupdate_scatter(cnt_f, [d16], ov)
            pltpu.make_async_copy(rows[b], agg_sh.at[idxs[b]], aas[b]).wait()
            if not last:
                @pl.when(i + 2 < steps)
                def _():
                    fstart(i + 2, b)

        fstart(0, 0)
        fstart(1, 1)
        runs(0, 0, last=False)
        runs(1, 1, last=False)

        @pl.loop(1, steps // 2)
        def _(p):
            for b in (0, 1):
                runs(2 * p + b, b, last=False)

        if steps % 2:
            runs(steps - 1, 0, last=True)

        @pl.loop(0, CR)
        def _(r):
            for j in range(128 // 16):
                cnt_l[r, pl.ds(j * 16, 16)] = cnt_f[pl.ds(r * 128 + j * 16, 16)]

        pltpu.sync_copy(zidx_hbm.at[pl.ds(0, CR)], idx_w)
        pltpu.sync_copy(cnt_l, cnt_sh.at[idx_w], add=True)

        plsc.subcore_barrier()

        @pl.loop(0, ZR // ZB)
        def _(t):
            base = sid * ZR + t * ZB
            pltpu.sync_copy(zidx_hbm.at[pl.ds(base, ZB)], idx_w)
            pltpu.sync_copy(agg_sh.at[idx_w], rows0)
            pltpu.sync_copy(rows0, aggp_hbm.at[pl.ds(cid * NPAD + base, ZB)])

        @pl.when(sid == 0)
        def _():
            pltpu.sync_copy(zidx_hbm.at[pl.ds(0, CR)], idx_w)
            pltpu.sync_copy(cnt_sh.at[idx_w], cnt_l)
            pltpu.sync_copy(cnt_l, cntp_hbm.at[pl.ds(cid * CR, CR)])

    return k(msg, dst, zidx)


def _tc_norm(h, aggp, cntp, gamma, beta):
    N, D = h.shape
    BN = 2000
    CW = cntp.shape[-1]

    def body(h_ref, a_ref, c_ref, g_ref, b_ref, o_ref):
        agg = a_ref[0] + a_ref[1]
        cnt = c_ref[0] + c_ref[1] + 1.0
        y = h_ref[...] + agg / cnt
        mean = jnp.mean(y, axis=-1, keepdims=True)
        var = jnp.mean((y - mean) ** 2, axis=-1, keepdims=True)
        o_ref[...] = (y - mean) * lax.rsqrt(var + 1e-5) * g_ref[...] + b_ref[...]

    return pl.pallas_call(
        body,
        grid=(N // BN,),
        in_specs=[
            pl.BlockSpec((BN, D), lambda i: (i, 0)),
            pl.BlockSpec((NC, BN, D), lambda i: (0, i, 0)),
            pl.BlockSpec((NC, BN, CW), lambda i: (0, i, 0)),
            pl.BlockSpec((1, D), lambda i: (0, 0)),
            pl.BlockSpec((1, D), lambda i: (0, 0)),
        ],
        out_specs=pl.BlockSpec((BN, D), lambda i: (i, 0)),
        out_shape=jax.ShapeDtypeStruct((N, D), jnp.float32),
    )(h, aggp, cntp, gamma, beta)


def kernel(h, edge_index, edge_attr, W1, b1, W2, b2, gamma, beta):
    N, D = h.shape
    src = edge_index[0]
    dst = edge_index[1]
    W1a = W1[:D]
    W1b = W1[D:]
    eaT = edge_attr.T
    b1r = b1.reshape(1, D)
    b2r = b2.reshape(1, D)

    hW = _tc_premul(h, W1a)
    gathered = _sc_gather(hW, src)
    msg = _tc_mlp(gathered, eaT, W1b, b1r, W2, b2r)
    zidx = jnp.arange(10240, dtype=jnp.int32)
    aggp, cntp = _sc_scatter(msg, dst, zidx, N)
    aggp = aggp.reshape(NC, -1, D)[:, :N]
    cntp = cntp.reshape(NC, -1)[:, :N].reshape(NC, N, 1)
    return _tc_norm(h, aggp, cntp, gamma.reshape(1, D), beta.reshape(1, D))


# speedup vs baseline: 4.6194x; 1.0003x over previous
import functools

import jax
import jax.numpy as jnp
from jax import lax
from jax.experimental import pallas as pl
from jax.experimental.pallas import tpu as pltpu
from jax.experimental.pallas import tpu_sc as plsc

NC = 2
NS = 16
NW = NC * NS
CH = 80


def _tc_premul(h, W1a):
    N, D = h.shape
    BN = 2000

    def body(h_ref, w_ref, o_ref):
        o_ref[...] = jnp.dot(h_ref[...].astype(jnp.bfloat16),
                             w_ref[...].astype(jnp.bfloat16),
                             preferred_element_type=jnp.float32)

    return pl.pallas_call(
        body,
        grid=(N // BN,),
        in_specs=[
            pl.BlockSpec((BN, D), lambda i: (i, 0)),
            pl.BlockSpec((D, D), lambda i: (0, 0)),
        ],
        out_specs=pl.BlockSpec((BN, D), lambda i: (i, 0)),
        out_shape=jax.ShapeDtypeStruct((N, D), jnp.float32),
    )(h, W1a)


def _sc_gather(hW, src):
    N, D = hW.shape
    E = src.shape[0]
    EPW = E // NW
    steps = EPW // CH
    mesh = plsc.VectorSubcoreMesh(core_axis_name="c", subcore_axis_name="s")

    @functools.partial(
        pl.kernel,
        out_type=jax.ShapeDtypeStruct((E, D), jnp.float32),
        mesh=mesh,
        scratch_types=[
            pltpu.VMEM((CH,), jnp.int32),
            pltpu.VMEM((CH,), jnp.int32),
            pltpu.VMEM((CH, D), jnp.float32),
            pltpu.VMEM((CH, D), jnp.float32),
            pltpu.SemaphoreType.DMA,
            pltpu.SemaphoreType.DMA,
            pltpu.SemaphoreType.DMA,
            pltpu.SemaphoreType.DMA,
            pltpu.SemaphoreType.DMA,
            pltpu.SemaphoreType.DMA,
        ],
    )
    def k(hW_hbm, src_hbm, out_hbm, idx0, idx1, rows0, rows1,
          si0, si1, sg0, sg1, so0, so1):
        cid = lax.axis_index("c")
        sid = lax.axis_index("s")
        wid = sid * NC + cid
        idxs = (idx0, idx1)
        rows = (rows0, rows1)
        isem = (si0, si1)
        gsem = (sg0, sg1)
        osem = (so0, so1)

        def istart(i, b):
            base = wid * EPW + i * CH
            pltpu.async_copy(src_hbm.at[pl.ds(base, CH)], idxs[b], isem[b])

        def iwait(b):
            pltpu.make_async_copy(src_hbm.at[pl.ds(0, CH)], idxs[b],
                                  isem[b]).wait()

        def owait(b):
            pltpu.make_async_copy(rows[b], out_hbm.at[pl.ds(0, CH)],
                                  osem[b]).wait()

        def run(i, b, first, last):
            iwait(b)
            if not first:
                owait(b)
            pltpu.async_copy(hW_hbm.at[idxs[b]], rows[b], gsem[b]).wait()
            base = wid * EPW + i * CH
            pltpu.async_copy(rows[b], out_hbm.at[pl.ds(base, CH)], osem[b])
            if not last:
                @pl.when(i + 2 < steps)
                def _():
                    istart(i + 2, b)

        istart(0, 0)
        istart(1, 1)
        run(0, 0, first=True, last=False)
        run(1, 1, first=True, last=False)

        @pl.loop(1, steps // 2)
        def _(p):
            for b in (0, 1):
                run(2 * p + b, b, first=False, last=False)

        if steps % 2:
            run(steps - 1, 0, first=False, last=True)
        owait(steps % 2)
        owait(1 - steps % 2)

    return k(hW, src)


def _tc_mlp(gathered, eaT, W1b, b1, W2, b2):
    E, D = gathered.shape
    F = eaT.shape[0]
    BE = 2560

    def body(g_ref, ea_ref, w1b_ref, b1_ref, w2_ref, b2_ref, o_ref):
        pre = lax.dot_general(ea_ref[...].astype(jnp.bfloat16),
                              w1b_ref[...].astype(jnp.bfloat16),
                              (((0,), (0,)), ((), ())),
                              preferred_element_type=jnp.float32)
        x = jnp.maximum(g_ref[...] + pre + b1_ref[...], 0.0)
        o_ref[...] = jnp.dot(x.astype(jnp.bfloat16),
                             w2_ref[...].astype(jnp.bfloat16),
                             preferred_element_type=jnp.float32) + b2_ref[...]

    return pl.pallas_call(
        body,
        grid=(E // BE,),
        in_specs=[
            pl.BlockSpec((BE, D), lambda i: (i, 0)),
            pl.BlockSpec((F, BE), lambda i: (0, i)),
            pl.BlockSpec((F, D), lambda i: (0, 0)),
            pl.BlockSpec((1, D), lambda i: (0, 0)),
            pl.BlockSpec((D, D), lambda i: (0, 0)),
            pl.BlockSpec((1, D), lambda i: (0, 0)),
        ],
        out_specs=pl.BlockSpec((BE, D), lambda i: (i, 0)),
        out_shape=jax.ShapeDtypeStruct((E, D), jnp.float32),
    )(gathered, eaT, W1b, b1, W2, b2)


def _sc_scatter(msg, dst, zidx, N):
    E, D = msg.shape
    EPW = E // NW
    steps = EPW // CH
    NPAD = 10240
    CR = NPAD // 128
    ZR = NPAD // NS
    ZB = 80
    mesh = plsc.VectorSubcoreMesh(core_axis_name="c", subcore_axis_name="s")

    @functools.partial(
        pl.kernel,
        out_type=(
            jax.ShapeDtypeStruct((NC * NPAD, D), jnp.float32),
            jax.ShapeDtypeStruct((NC * CR, 128), jnp.float32),
        ),
        mesh=mesh,
        compiler_params=pltpu.CompilerParams(needs_layout_passes=False),
        scratch_types=[
            pltpu.VMEM_SHARED((NPAD, D), jnp.float32),
            pltpu.VMEM_SHARED((CR, 128), jnp.float32),
            pltpu.VMEM((CH,), jnp.int32),
            pltpu.VMEM((CH,), jnp.int32),
            pltpu.VMEM((CH, D), jnp.float32),
            pltpu.VMEM((CH, D), jnp.float32),
            pltpu.VMEM((ZB,), jnp.int32),
            pltpu.VMEM((NPAD,), jnp.float32),
            pltpu.VMEM((CR, 128), jnp.float32),
            pltpu.SemaphoreType.DMA,
            pltpu.SemaphoreType.DMA,
            pltpu.SemaphoreType.DMA,
            pltpu.SemaphoreType.DMA,
            pltpu.SemaphoreType.DMA,
            pltpu.SemaphoreType.DMA,
            pltpu.SemaphoreType.DMA,
        ],
    )
    def k(msg_hbm, dst_hbm, zidx_hbm, aggp_hbm, cntp_hbm,
          agg_sh, cnt_sh, idx0, idx1, rows0, rows1, idx_w, cnt_f,
          cnt_l, sem, ii0, ii1, mm0, mm1, aa0, aa1):
        cid = lax.axis_index("c")
        sid = lax.axis_index("s")
        wid = sid * NC + cid
        zv = jnp.zeros((16,), jnp.float32)
        ov = jnp.full((16,), 1.0, jnp.float32)

        @pl.loop(0, ZB)
        def _(r):
            for j in range(D // 16):
                rows0[r, pl.ds(j * 16, 16)] = zv

        @pl.loop(0, CR)
        def _(r):
            for j in range(128 // 16):
                cnt_f[pl.ds(r * 128 + j * 16, 16)] = zv

        @pl.loop(0, ZR // ZB)
        def _(t):
            base = sid * ZR + t * ZB
            pltpu.sync_copy(zidx_hbm.at[pl.ds(base, ZB)], idx_w)
            pltpu.sync_copy(rows0, agg_sh.at[idx_w])

        @pl.when(sid == 0)
        def _():
            pltpu.sync_copy(zidx_hbm.at[pl.ds(0, ZB)], idx_w)
            pltpu.sync_copy(rows0, cnt_sh.at[idx_w])

        plsc.subcore_barrier()

        idxs = (idx0, idx1)
        rows = (rows0, rows1)
        iis = (ii0, ii1)
        mms = (mm0, mm1)
        aas = (aa0, aa1)

        def fstart(i, b):
            base = wid * EPW + i * CH
            pltpu.async_copy(dst_hbm.at[pl.ds(base, CH)], idxs[b], iis[b])
            pltpu.async_copy(msg_hbm.at[pl.ds(base, CH)], rows[b], mms[b])

        def runs(i, b, last):
            pltpu.make_async_copy(dst_hbm.at[pl.ds(0, CH)], idxs[b],
                                  iis[b]).wait()
            pltpu.make_async_copy(msg_hbm.at[pl.ds(0, CH)], rows[b],
                                  mms[b]).wait()
            pltpu.async_copy(rows[b], agg_sh.at[idxs[b]], aas[b], add=True)
            for kk in range(CH // 16):
                d16 = idxs[b][pl.ds(kk * 16, 16)]
                plsc.addupdate_scatter(cnt_f, [d16], ov)
            pltpu.make_async_copy(rows[b], agg_sh.at[idxs[b]], aas[b]).wait()
            if not last:
                @pl.when(i + 2 < steps)
                def _():
                    fstart(i + 2, b)

        fstart(0, 0)
        fstart(1, 1)
        runs(0, 0, last=False)
        runs(1, 1, last=False)

        @pl.loop(1, steps // 2)
        def _(p):
            for b in (0, 1):
                runs(2 * p + b, b, last=False)

        if steps % 2:
            runs(steps - 1, 0, last=True)

        @pl.loop(0, CR)
        def _(r):
            for j in range(128 // 16):
                cnt_l[r, pl.ds(j * 16, 16)] = cnt_f[pl.ds(r * 128 + j * 16, 16)]

        pltpu.sync_copy(zidx_hbm.at[pl.ds(0, CR)], idx_w)
        pltpu.sync_copy(cnt_l, cnt_sh.at[idx_w], add=True)

        plsc.subcore_barrier()

        @pl.loop(0, ZR // ZB)
        def _(t):
            base = sid * ZR + t * ZB
            pltpu.sync_copy(zidx_hbm.at[pl.ds(base, ZB)], idx_w)
            pltpu.sync_copy(agg_sh.at[idx_w], rows0)
            pltpu.sync_copy(rows0, aggp_hbm.at[pl.ds(cid * NPAD + base, ZB)])

        @pl.when(sid == 0)
        def _():
            pltpu.sync_copy(zidx_hbm.at[pl.ds(0, CR)], idx_w)
            pltpu.sync_copy(cnt_sh.at[idx_w], cnt_l)
            pltpu.sync_copy(cnt_l, cntp_hbm.at[pl.ds(cid * CR, CR)])

    return k(msg, dst, zidx)


def _tc_norm(h, aggp, cntp, gamma, beta):
    N, D = h.shape
    BN = 2000
    CW = cntp.shape[-1]

    def body(h_ref, a_ref, c_ref, g_ref, b_ref, o_ref):
        agg = a_ref[0] + a_ref[1]
        cnt = c_ref[0] + c_ref[1] + 1.0
        y = h_ref[...] + agg / cnt
        mean = jnp.mean(y, axis=-1, keepdims=True)
        var = jnp.mean((y - mean) ** 2, axis=-1, keepdims=True)
        o_ref[...] = (y - mean) * lax.rsqrt(var + 1e-5) * g_ref[...] + b_ref[...]

    return pl.pallas_call(
        body,
        grid=(N // BN,),
        in_specs=[
            pl.BlockSpec((BN, D), lambda i: (i, 0)),
            pl.BlockSpec((NC, BN, D), lambda i: (0, i, 0)),
            pl.BlockSpec((NC, BN, CW), lambda i: (0, i, 0)),
            pl.BlockSpec((1, D), lambda i: (0, 0)),
            pl.BlockSpec((1, D), lambda i: (0, 0)),
        ],
        out_specs=pl.BlockSpec((BN, D), lambda i: (i, 0)),
        out_shape=jax.ShapeDtypeStruct((N, D), jnp.float32),
    )(h, aggp, cntp, gamma, beta)


def kernel(h, edge_index, edge_attr, W1, b1, W2, b2, gamma, beta):
    N, D = h.shape
    src = edge_index[0]
    dst = edge_index[1]
    W1a = W1[:D]
    W1b = W1[D:]
    eaT = edge_attr.T
    b1r = b1.reshape(1, D)
    b2r = b2.reshape(1, D)

    hW = _tc_premul(h, W1a)
    gathered = _sc_gather(hW, src)
    msg = _tc_mlp(gathered, eaT, W1b, b1r, W2, b2r)
    zidx = jnp.arange(10240, dtype=jnp.int32)
    aggp, cntp = _sc_scatter(msg, dst, zidx, N)
    aggp = aggp.reshape(NC, -1, D)[:, :N]
    cntp = cntp.reshape(NC, -1)[:, :N].reshape(NC, N, 1)
    return _tc_norm(h, aggp, cntp, gamma.reshape(1, D), beta.reshape(1, D))
